# baseline (speedup 1.0000x reference)
import functools

import jax
import jax.numpy as jnp
from jax import lax
from jax.experimental import pallas as pl
from jax.experimental.pallas import tpu as pltpu
from jax.experimental.pallas import tpu_sc as plsc

NUM_CARDS = 1000000
EMBED_DIM = 32
BATCH = 16384
HIST = 200

NW = 32
BB = BATCH // 128
BB_PER_W = BB // NW
NBUF = 4
NSLAB = BB * HIST // 8


def _gather_kernel(idsT_hbm, table_hbm, out_hbm, idx_v, rows_v, sem_i, sem_g, sem_o):
    wid = lax.axis_index("s") * 2 + lax.axis_index("c")
    bb0 = wid * BB_PER_W

    def idx_copy(h, u, s):
        return pltpu.make_async_copy(
            idsT_hbm.at[h, pl.ds((bb0 + u) * 128, 128)], idx_v.at[s], sem_i.at[s])

    def gather_copy(s):
        return pltpu.make_async_copy(
            table_hbm.at[idx_v.at[s]], rows_v.at[s], sem_g.at[s])

    def out_copy(h, u, s):
        slab = (bb0 + u) * (HIST // 8) + h // 8
        return pltpu.make_async_copy(
            rows_v.at[s],
            out_hbm.at[slab, :, pl.ds((h % 8) * EMBED_DIM, EMBED_DIM)],
            sem_o.at[s])

    idx_copy(0, 0, 0).start()

    def body(h, carry):
        for u in range(NBUF):
            @pl.when(h >= 1)
            def _():
                out_copy(h - 1, u, u).wait()

            idx_copy(h, u, u).wait()
            gather_copy(u).start()

            prev = (u - 1) % NBUF
            if u > 0:
                gather_copy(prev).wait()
                out_copy(h, u - 1, prev).start()
            else:
                @pl.when(h >= 1)
                def _():
                    gather_copy(prev).wait()
                    out_copy(h - 1, NBUF - 1, prev).start()

            if u < NBUF - 1:
                idx_copy(h, u + 1, u + 1).start()
            else:
                @pl.when(h < HIST - 1)
                def _():
                    idx_copy(h + 1, 0, 0).start()
        return carry

    lax.fori_loop(0, HIST, body, 0)

    gather_copy(NBUF - 1).wait()
    out_copy(HIST - 1, NBUF - 1, NBUF - 1).start()
    for u in range(NBUF):
        out_copy(HIST - 1, u, u).wait()


def _relayout_kernel(x_ref, y_ref):
    x = x_ref[...]
    j = lax.broadcasted_iota(jnp.int32, (256, 128), 0)
    b2 = lax.broadcasted_iota(jnp.int32, (256, 128), 1)
    dn = (((0,), (0,)), ((), ()))
    halves = []
    for chalf in (0, 1):
        e = (j == 2 * b2 + chalf).astype(jnp.float32)
        halves.append(lax.dot_general(x, e, dn,
                                      preferred_element_type=jnp.float32))
    y = jnp.concatenate(halves, axis=0)
    y_ref[...] = y.reshape(8, 4, 1, 8, 128)


@jax.jit
def _embed(idsT, table):
    gather = functools.partial(
        pl.kernel,
        out_type=jax.ShapeDtypeStruct((NSLAB, 128, 8 * EMBED_DIM), jnp.float32),
        mesh=plsc.VectorSubcoreMesh(core_axis_name="c", subcore_axis_name="s"),
        scratch_types=[
            pltpu.VMEM((NBUF, 128), jnp.int32),
            pltpu.VMEM((NBUF, 128, EMBED_DIM), jnp.float32),
            pltpu.SemaphoreType.DMA((NBUF,)),
            pltpu.SemaphoreType.DMA((NBUF,)),
            pltpu.SemaphoreType.DMA((NBUF,)),
        ],
        compiler_params=pltpu.CompilerParams(use_tc_tiling_on_sc=False),
    )(_gather_kernel)
    slabs = gather(idsT, table)
    in128 = slabs.reshape(NSLAB * 256, 128)

    out5 = pl.pallas_call(
        _relayout_kernel,
        grid=(BB, HIST // 8),
        in_specs=[pl.BlockSpec((256, 128), lambda j, i: (j * 25 + i, 0))],
        out_specs=pl.BlockSpec((8, 4, 1, 8, 128), lambda j, i: (i, 0, j, 0, 0)),
        out_shape=jax.ShapeDtypeStruct((HIST, 4, BB, 8, 128), jnp.float32),
    )(in128)
    return out5


def kernel(ids, table):
    idsT = ids.astype(jnp.int32).T
    out5 = _embed(idsT, table)
    t = out5.transpose(2, 4, 0, 1, 3)
    return t.reshape(BATCH, HIST, EMBED_DIM)


# speedup vs baseline: 1.2952x; 1.2952x over previous
import functools

import jax
import jax.numpy as jnp
from jax import lax
from jax.experimental import pallas as pl
from jax.experimental.pallas import tpu as pltpu
from jax.experimental.pallas import tpu_sc as plsc

NUM_CARDS = 1000000
EMBED_DIM = 32
BATCH = 16384
HIST = 200

NW = 32
ROWS_PER_W = BATCH // NW
K = 4
NBUF = 4
STEPS = ROWS_PER_W // K
OUTER = STEPS // NBUF
SPLITS = ((0, 128), (128, 72))


def _embed_kernel(ids_hbm, table_hbm, out_hbm, idx_v, rows_v, sem_i, sem_g, sem_o):
    wid = lax.axis_index("s") * 2 + lax.axis_index("c")
    base0 = wid * ROWS_PER_W

    def idx_copy(g, b):
        return pltpu.make_async_copy(
            ids_hbm.at[pl.ds(base0 + g * K, K)], idx_v.at[b], sem_i.at[b])

    def out_copy(g, b):
        return pltpu.make_async_copy(
            rows_v.at[b], out_hbm.at[pl.ds(base0 + g * K, K)], sem_o.at[b])

    def gathers(b):
        for j in range(K):
            for off, n in SPLITS:
                yield pltpu.make_async_copy(
                    table_hbm.at[idx_v.at[b].at[j].at[pl.ds(off, n)]],
                    rows_v.at[b].at[j].at[pl.ds(off, n)],
                    sem_g.at[b])

    def drain_and_flush(g, b):
        for c in gathers(b):
            c.wait()
        out_copy(g, b).start()

    idx_copy(0, 0).start()

    def outer(p, carry):
        for u in range(NBUF):
            g = p * NBUF + u
            prev = (u - 1) % NBUF

            @pl.when(p >= 1)
            def _():
                out_copy(g - NBUF, u).wait()

            idx_copy(g, u).wait()
            for c in gathers(u):
                c.start()

            if u > 0:
                drain_and_flush(g - 1, prev)
            else:
                @pl.when(p >= 1)
                def _():
                    drain_and_flush(g - 1, prev)

            if u < NBUF - 1:
                idx_copy(g + 1, u + 1).start()
            else:
                @pl.when(p < OUTER - 1)
                def _():
                    idx_copy(g + 1, 0).start()
        return carry

    lax.fori_loop(0, OUTER, outer, 0)

    drain_and_flush(STEPS - 1, NBUF - 1)
    for b in range(NBUF):
        out_copy(STEPS - NBUF + b, b).wait()


@jax.jit
def _embed(ids, table):
    fn = functools.partial(
        pl.kernel,
        out_type=jax.ShapeDtypeStruct((BATCH, HIST, EMBED_DIM), jnp.float32),
        mesh=plsc.VectorSubcoreMesh(core_axis_name="c", subcore_axis_name="s"),
        scratch_types=[
            pltpu.VMEM((NBUF, K, HIST), jnp.int32),
            pltpu.VMEM((NBUF, K, HIST, EMBED_DIM), jnp.float32),
            pltpu.SemaphoreType.DMA((NBUF,)),
            pltpu.SemaphoreType.DMA((NBUF,)),
            pltpu.SemaphoreType.DMA((NBUF,)),
        ],
        compiler_params=pltpu.CompilerParams(use_tc_tiling_on_sc=False),
    )(_embed_kernel)
    return fn(ids, table)


def kernel(ids, table):
    return _embed(ids.astype(jnp.int32), table)
